# baseline (speedup 1.0000x reference)
import functools

import jax
import jax.numpy as jnp
from jax.experimental import pallas as pl
from jax.experimental.pallas import tpu as pltpu

RV = 256
BIG = 1e30
WFLOOR = 1e-37


def _gravnet_block(x_all_ref, ws_ref, bs_ref, wf_ref, bf_ref, wo_ref, bo_ref,
                   out_ref, *, n_neigh, v_total):
    r = pl.program_id(1)
    x_all = x_all_ref[0]

    f_all_t = jax.lax.dot_general(
        wf_ref[...], x_all, (((0,), (1,)), ((), ())),
        preferred_element_type=jnp.float32) + bf_ref[...][:, None]

    c_all_t = jax.lax.dot_general(
        ws_ref[...], x_all, (((0,), (1,)), ((), ())),
        preferred_element_type=jnp.float32) + bs_ref[...][:, None]

    r0 = r * RV
    x_row = x_all_ref[0, pl.ds(r0, RV), :]
    c_row = jnp.dot(x_row, ws_ref[...],
                    preferred_element_type=jnp.float32) + bs_ref[...]

    n_dim = c_row.shape[1]
    dist = jnp.zeros((RV, v_total), dtype=jnp.float32)
    for d in range(n_dim):
        t = c_row[:, d:d + 1] - c_all_t[d:d + 1, :]
        dist = dist + t * t

    col_ids = jax.lax.broadcasted_iota(jnp.int32, (RV, v_total), 1)
    row_ids = jax.lax.broadcasted_iota(jnp.int32, (RV, v_total), 0) + r0
    dist = jnp.where(col_ids == row_ids, BIG, dist)

    k = n_neigh - 1
    n_prop = f_all_t.shape[0]

    def body(_, carry):
        d, wacc = carry
        m = jnp.min(d, axis=1)
        w = jnp.maximum(jnp.exp(-m), WFLOOR)
        eqf = (d == m[:, None]).astype(jnp.float32)
        d = d + eqf * BIG
        wacc = wacc + eqf * w[:, None]
        return d, wacc

    w0 = jnp.zeros((RV, v_total), dtype=jnp.float32)
    _, wacc = jax.lax.fori_loop(0, k, body, (dist, w0))

    sm = jax.lax.dot_general(
        wacc, f_all_t, (((1,), (1,)), ((), ())),
        preferred_element_type=jnp.float32) / float(k)

    sel = wacc > 0.0
    cols = []
    for p in range(n_prop):
        prod = wacc * f_all_t[p:p + 1, :]
        cols.append(jnp.max(jnp.where(sel, prod, -BIG), axis=1)[:, None])
    mx = jnp.concatenate(cols, axis=1)

    cat = jnp.concatenate([x_row, mx, sm], axis=1)
    out = jnp.dot(cat, wo_ref[...], preferred_element_type=jnp.float32)
    out_ref[0] = jnp.tanh(out + bo_ref[...])


def kernel(x, Ws, bs, Wf, bf, Wo, bo):
    b, v, f_in = x.shape
    n_neigh = 40
    n_filters = Wo.shape[1]
    grid = (b, v // RV)

    body = functools.partial(_gravnet_block, n_neigh=n_neigh, v_total=v)
    return pl.pallas_call(
        body,
        grid=grid,
        in_specs=[
            pl.BlockSpec((1, v, f_in), lambda bi, ri: (bi, 0, 0)),
            pl.BlockSpec(Ws.shape, lambda bi, ri: (0, 0)),
            pl.BlockSpec(bs.shape, lambda bi, ri: (0,)),
            pl.BlockSpec(Wf.shape, lambda bi, ri: (0, 0)),
            pl.BlockSpec(bf.shape, lambda bi, ri: (0,)),
            pl.BlockSpec(Wo.shape, lambda bi, ri: (0, 0)),
            pl.BlockSpec(bo.shape, lambda bi, ri: (0,)),
        ],
        out_specs=pl.BlockSpec((1, RV, n_filters), lambda bi, ri: (bi, ri, 0)),
        out_shape=jax.ShapeDtypeStruct((b, v, n_filters), jnp.float32),
    )(x, Ws, bs, Wf, bf, Wo, bo)


# speedup vs baseline: 2.3061x; 2.3061x over previous
import functools

import jax
import jax.numpy as jnp
from jax.experimental import pallas as pl
from jax.experimental.pallas import tpu as pltpu

RV = 256
BIG = 1e30


def _gravnet_block(x_all_ref, ws_ref, bs_ref, wf_ref, bf_ref, wo_ref, bo_ref,
                   out_ref, *, n_neigh, v_total):
    r = pl.program_id(1)
    x_all = x_all_ref[0]

    f_all_t = jax.lax.dot_general(
        wf_ref[...], x_all, (((0,), (1,)), ((), ())),
        preferred_element_type=jnp.float32) + bf_ref[...][:, None]

    c_all_t = jax.lax.dot_general(
        ws_ref[...], x_all, (((0,), (1,)), ((), ())),
        preferred_element_type=jnp.float32) + bs_ref[...][:, None]

    r0 = r * RV
    x_row = x_all_ref[0, pl.ds(r0, RV), :]
    c_row = jnp.dot(x_row, ws_ref[...],
                    preferred_element_type=jnp.float32) + bs_ref[...]

    n_dim = c_row.shape[1]
    dist = jnp.zeros((RV, v_total), dtype=jnp.float32)
    for d in range(n_dim):
        t = c_row[:, d:d + 1] - c_all_t[d:d + 1, :]
        dist = dist + t * t

    col_ids = jax.lax.broadcasted_iota(jnp.int32, (RV, v_total), 1)
    row_ids = jax.lax.broadcasted_iota(jnp.int32, (RV, v_total), 0) + r0
    dist = jnp.where(col_ids == row_ids, BIG, dist)

    k = n_neigh - 1
    n_prop = f_all_t.shape[0]
    f_all_t_bf = f_all_t.astype(jnp.bfloat16)

    def body(_, carry):
        d, mx, sm = carry
        m = jnp.min(d, axis=1)
        eq = d == m[:, None]
        eqf = eq.astype(jnp.bfloat16)
        d = jnp.where(eq, BIG, d)
        feat = jax.lax.dot_general(
            eqf, f_all_t_bf, (((1,), (1,)), ((), ())),
            preferred_element_type=jnp.float32)
        wf = jnp.exp(-m)[:, None] * feat
        return d, jnp.maximum(mx, wf), sm + wf

    mx0 = jnp.full((RV, n_prop), -BIG, dtype=jnp.float32)
    sm0 = jnp.zeros((RV, n_prop), dtype=jnp.float32)
    _, mx, sm = jax.lax.fori_loop(0, k, body, (dist, mx0, sm0),
                                  unroll=2)

    cat = jnp.concatenate([x_row, mx, sm / float(k)], axis=1)
    out = jnp.dot(cat, wo_ref[...], preferred_element_type=jnp.float32)
    out_ref[0] = jnp.tanh(out + bo_ref[...])


def kernel(x, Ws, bs, Wf, bf, Wo, bo):
    b, v, f_in = x.shape
    n_neigh = 40
    n_filters = Wo.shape[1]
    grid = (b, v // RV)

    body = functools.partial(_gravnet_block, n_neigh=n_neigh, v_total=v)
    return pl.pallas_call(
        body,
        grid=grid,
        in_specs=[
            pl.BlockSpec((1, v, f_in), lambda bi, ri: (bi, 0, 0)),
            pl.BlockSpec(Ws.shape, lambda bi, ri: (0, 0)),
            pl.BlockSpec(bs.shape, lambda bi, ri: (0,)),
            pl.BlockSpec(Wf.shape, lambda bi, ri: (0, 0)),
            pl.BlockSpec(bf.shape, lambda bi, ri: (0,)),
            pl.BlockSpec(Wo.shape, lambda bi, ri: (0, 0)),
            pl.BlockSpec(bo.shape, lambda bi, ri: (0,)),
        ],
        out_specs=pl.BlockSpec((1, RV, n_filters), lambda bi, ri: (bi, ri, 0)),
        out_shape=jax.ShapeDtypeStruct((b, v, n_filters), jnp.float32),
    )(x, Ws, bs, Wf, bf, Wo, bo)


# speedup vs baseline: 3.0013x; 1.3014x over previous
import functools

import jax
import jax.numpy as jnp
from jax.experimental import pallas as pl
from jax.experimental.pallas import tpu as pltpu

RV = 256
BIG = 1e30


def _gravnet_block(x_all_ref, ws_ref, bs_ref, wf_ref, bf_ref, wo_ref, bo_ref,
                   out_ref, *, n_neigh, v_total):
    r = pl.program_id(1)
    x_all = x_all_ref[0]

    f_all_t = jax.lax.dot_general(
        wf_ref[...], x_all, (((0,), (1,)), ((), ())),
        preferred_element_type=jnp.float32) + bf_ref[...][:, None]

    c_all_t = jax.lax.dot_general(
        ws_ref[...], x_all, (((0,), (1,)), ((), ())),
        preferred_element_type=jnp.float32) + bs_ref[...][:, None]

    r0 = r * RV
    x_row = x_all_ref[0, pl.ds(r0, RV), :]
    c_row = jnp.dot(x_row, ws_ref[...],
                    preferred_element_type=jnp.float32) + bs_ref[...]

    n_dim = c_row.shape[1]
    dist = jnp.zeros((RV, v_total), dtype=jnp.float32)
    for d in range(n_dim):
        t = c_row[:, d:d + 1] - c_all_t[d:d + 1, :]
        dist = dist + t * t

    col_ids = jax.lax.broadcasted_iota(jnp.int32, (RV, v_total), 1)
    row_ids = jax.lax.broadcasted_iota(jnp.int32, (RV, v_total), 0) + r0
    dist = jnp.where(col_ids == row_ids, BIG, dist)

    k = n_neigh - 1
    n_prop = f_all_t.shape[0]
    f_all_t_bf = f_all_t.astype(jnp.bfloat16)

    def body(_, carry):
        d, m, mx, sm = carry
        eq = d == m[:, None]
        eqf = eq.astype(jnp.bfloat16)
        d = jnp.where(eq, BIG, d)
        m_next = jnp.min(d, axis=1)
        feat = jax.lax.dot_general(
            eqf, f_all_t_bf, (((1,), (1,)), ((), ())),
            preferred_element_type=jnp.float32)
        wf = jnp.exp(-m)[:, None] * feat
        return d, m_next, jnp.maximum(mx, wf), sm + wf

    mx0 = jnp.full((RV, n_prop), -BIG, dtype=jnp.float32)
    sm0 = jnp.zeros((RV, n_prop), dtype=jnp.float32)
    m0 = jnp.min(dist, axis=1)
    _, _, mx, sm = jax.lax.fori_loop(0, k, body, (dist, m0, mx0, sm0),
                                     unroll=4)

    cat = jnp.concatenate([x_row, mx, sm / float(k)], axis=1)
    out = jnp.dot(cat, wo_ref[...], preferred_element_type=jnp.float32)
    out_ref[0] = jnp.tanh(out + bo_ref[...])


def kernel(x, Ws, bs, Wf, bf, Wo, bo):
    b, v, f_in = x.shape
    n_neigh = 40
    n_filters = Wo.shape[1]
    grid = (b, v // RV)

    body = functools.partial(_gravnet_block, n_neigh=n_neigh, v_total=v)
    return pl.pallas_call(
        body,
        grid=grid,
        in_specs=[
            pl.BlockSpec((1, v, f_in), lambda bi, ri: (bi, 0, 0)),
            pl.BlockSpec(Ws.shape, lambda bi, ri: (0, 0)),
            pl.BlockSpec(bs.shape, lambda bi, ri: (0,)),
            pl.BlockSpec(Wf.shape, lambda bi, ri: (0, 0)),
            pl.BlockSpec(bf.shape, lambda bi, ri: (0,)),
            pl.BlockSpec(Wo.shape, lambda bi, ri: (0, 0)),
            pl.BlockSpec(bo.shape, lambda bi, ri: (0,)),
        ],
        out_specs=pl.BlockSpec((1, RV, n_filters), lambda bi, ri: (bi, ri, 0)),
        out_shape=jax.ShapeDtypeStruct((b, v, n_filters), jnp.float32),
    )(x, Ws, bs, Wf, bf, Wo, bo)


# speedup vs baseline: 3.4043x; 1.1343x over previous
import functools

import jax
import jax.numpy as jnp
from jax.experimental import pallas as pl
from jax.experimental.pallas import tpu as pltpu

RV = 256
BIG = 1e30


def _gravnet_block(x_all_ref, ws_ref, bs_ref, wf_ref, bf_ref, wo_ref, bo_ref,
                   out_ref, *, n_neigh, v_total):
    r = pl.program_id(1)
    x_all = x_all_ref[0]

    f_all_t = jax.lax.dot_general(
        wf_ref[...], x_all, (((0,), (1,)), ((), ())),
        preferred_element_type=jnp.float32) + bf_ref[...][:, None]

    c_all_t = jax.lax.dot_general(
        ws_ref[...], x_all, (((0,), (1,)), ((), ())),
        preferred_element_type=jnp.float32) + bs_ref[...][:, None]

    r0 = r * RV
    x_row = x_all_ref[0, pl.ds(r0, RV), :]
    c_row = jnp.dot(x_row, ws_ref[...],
                    preferred_element_type=jnp.float32) + bs_ref[...]

    n_dim = c_row.shape[1]
    dist = jnp.zeros((RV, v_total), dtype=jnp.float32)
    for d in range(n_dim):
        t = c_row[:, d:d + 1] - c_all_t[d:d + 1, :]
        dist = dist + t * t

    col_ids = jax.lax.broadcasted_iota(jnp.int32, (RV, v_total), 1)
    row_ids = jax.lax.broadcasted_iota(jnp.int32, (RV, v_total), 0) + r0
    dist = jnp.where(col_ids == row_ids, BIG, dist)

    k = n_neigh - 1
    n_prop = f_all_t.shape[0]
    f_all_t_bf = f_all_t.astype(jnp.bfloat16)

    def body(_, carry):
        m, mx, sm = carry
        mcol = m[:, None]
        eqf = (dist == mcol).astype(jnp.bfloat16)
        m_next = jnp.min(jnp.where(dist > mcol, dist, BIG), axis=1)
        feat = jax.lax.dot_general(
            eqf, f_all_t_bf, (((1,), (1,)), ((), ())),
            preferred_element_type=jnp.float32)
        wf = jnp.exp(-m)[:, None] * feat
        return m_next, jnp.maximum(mx, wf), sm + wf

    mx0 = jnp.full((RV, n_prop), -BIG, dtype=jnp.float32)
    sm0 = jnp.zeros((RV, n_prop), dtype=jnp.float32)
    m0 = jnp.min(dist, axis=1)
    _, mx, sm = jax.lax.fori_loop(0, k, body, (m0, mx0, sm0),
                                  unroll=4)

    cat = jnp.concatenate([x_row, mx, sm / float(k)], axis=1)
    out = jnp.dot(cat, wo_ref[...], preferred_element_type=jnp.float32)
    out_ref[0] = jnp.tanh(out + bo_ref[...])


def kernel(x, Ws, bs, Wf, bf, Wo, bo):
    b, v, f_in = x.shape
    n_neigh = 40
    n_filters = Wo.shape[1]
    grid = (b, v // RV)

    body = functools.partial(_gravnet_block, n_neigh=n_neigh, v_total=v)
    return pl.pallas_call(
        body,
        grid=grid,
        in_specs=[
            pl.BlockSpec((1, v, f_in), lambda bi, ri: (bi, 0, 0)),
            pl.BlockSpec(Ws.shape, lambda bi, ri: (0, 0)),
            pl.BlockSpec(bs.shape, lambda bi, ri: (0,)),
            pl.BlockSpec(Wf.shape, lambda bi, ri: (0, 0)),
            pl.BlockSpec(bf.shape, lambda bi, ri: (0,)),
            pl.BlockSpec(Wo.shape, lambda bi, ri: (0, 0)),
            pl.BlockSpec(bo.shape, lambda bi, ri: (0,)),
        ],
        out_specs=pl.BlockSpec((1, RV, n_filters), lambda bi, ri: (bi, ri, 0)),
        out_shape=jax.ShapeDtypeStruct((b, v, n_filters), jnp.float32),
    )(x, Ws, bs, Wf, bf, Wo, bo)
